# Initial kernel scaffold; baseline (speedup 1.0000x reference)
import jax
import jax.numpy as jnp
from jax.experimental import pallas as pl


def kernel(query, time_ids, rel_table, k_len):
    raise NotImplementedError("write your pallas kernel here")


# speedup vs baseline: 34.7901x; 34.7901x over previous
import jax
import jax.numpy as jnp
from jax.experimental import pallas as pl
from jax.experimental.pallas import tpu as pltpu

_MAXREL = 16
_NIDX = 2 * _MAXREL + 1


_QC = 8


def _tc_body(tq_ref, q_ref, t_ref, tab_ref, o_ref):
    b = pl.program_id(0)
    qc = pl.program_id(1)
    tab = tab_ref[...]
    t_row = t_ref[0]
    Lk = t_row.shape[-1]
    iota = jax.lax.broadcasted_iota(jnp.int32, (_NIDX, Lk), 0)
    for j in range(_QC):
        qv = q_ref[0, j]
        scores = jax.lax.dot_general(
            qv, tab, (((1,), (1,)), ((), ())),
            preferred_element_type=jnp.float32)
        tq = tq_ref[b, qc * _QC + j]
        idx = jnp.clip(t_row - tq, -_MAXREL, _MAXREL) + _MAXREL
        onehot = jnp.where(iota == idx, 1.0, 0.0).astype(jnp.float32)
        out = jax.lax.dot_general(
            scores, onehot, (((1,), (0,)), ((), ())),
            preferred_element_type=jnp.float32)
        o_ref[0, :, j, :] = out


def kernel(query, time_ids, rel_table, k_len):
    B, H, Lq, Dh = query.shape
    Lk = time_ids.shape[1]
    start = k_len - Lk
    t = jax.lax.dynamic_slice_in_dim(time_ids, start, Lk, axis=1)
    t = t.astype(jnp.int32)
    tq = t[:, :Lq]
    t3 = t.reshape(B, 1, Lk)
    q_t = query.transpose(0, 2, 1, 3)

    out = pl.pallas_call(
        _tc_body,
        grid=(B, Lq // _QC),
        in_specs=[
            pl.BlockSpec(memory_space=pltpu.SMEM),
            pl.BlockSpec((1, _QC, H, Dh), lambda b, q: (b, q, 0, 0)),
            pl.BlockSpec((1, 1, Lk), lambda b, q: (b, 0, 0)),
            pl.BlockSpec((_NIDX, Dh), lambda b, q: (0, 0)),
        ],
        out_specs=pl.BlockSpec((1, H, _QC, Lk), lambda b, q: (b, 0, q, 0)),
        out_shape=jax.ShapeDtypeStruct((B, H, Lq, Lk), jnp.float32),
    )(tq, q_t, t3, rel_table)
    return out
